# baseline (speedup 1.0000x reference)
import jax
import jax.numpy as jnp
from jax import lax
from jax.experimental import pallas as pl
from jax.experimental.pallas import tpu as pltpu
from jax.experimental.pallas import tpu_sc as plsc

N_NODES = 10000
N_EDGES = 320000
D_FEAT = 128

NC = 2
NS = 16
NW = NC * NS

N_PAD = 10240
CHUNK = 40
NCH = 256
E_PER_W = NCH * CHUNK
E_PAD = NW * E_PER_W
NBUF = 8
NIDX = 8
ROWS_PER_TILE = N_PAD // NS
ZROWS = 16


def _sc_aggregate_body(x_hbm, ra_hbm, rb_hbm, out_hbm, *refs):
    ias = refs[0:NIDX]
    ibs = refs[NIDX:2 * NIDX]
    rws = refs[2 * NIDX:2 * NIDX + NBUF]
    zbuf = refs[2 * NIDX + NBUF]
    acc = refs[2 * NIDX + NBUF + 1]
    sis = refs[2 * NIDX + NBUF + 2:3 * NIDX + NBUF + 2]
    srs = refs[3 * NIDX + NBUF + 2:3 * NIDX + 2 * NBUF + 2]

    cid = lax.axis_index("c")
    sid = lax.axis_index("s")
    wid = cid * NS + sid
    ebase = wid * E_PER_W

    def idx_start(i, p):
        base = ebase + i * CHUNK
        pltpu.async_copy(ra_hbm.at[pl.ds(base, CHUNK)], ias[p], sis[p])
        pltpu.async_copy(rb_hbm.at[pl.ds(base, CHUNK)], ibs[p], sis[p])

    def idx_wait(p):
        pltpu.make_async_copy(ra_hbm.at[pl.ds(0, CHUNK)], ias[p], sis[p]).wait()
        pltpu.make_async_copy(rb_hbm.at[pl.ds(0, CHUNK)], ibs[p], sis[p]).wait()

    def rows_wait(p):
        pltpu.make_async_copy(x_hbm.at[pl.ds(0, CHUNK)], rws[p], srs[p]).wait()

    for p in range(NIDX):
        idx_start(p, p)

    with jax.named_scope("agg_zero_init"):
        for r in range(ZROWS):
            for j in range(8):
                zbuf[r, pl.ds(j * 16, 16)] = jnp.zeros((16,), jnp.float32)

        row0 = sid * ROWS_PER_TILE

        def zero_acc(k, _):
            pltpu.sync_copy(zbuf, acc.at[pl.ds(row0 + k * ZROWS, ZROWS)])
            return 0

        lax.fori_loop(0, ROWS_PER_TILE // ZROWS, zero_acc, 0)

        plsc.subcore_barrier()

    for j in range(NBUF):
        idx_wait(j)
        pltpu.async_copy(x_hbm.at[ias[j]], rws[j], srs[j])

    def step_block(k, _):
        for u in range(NIDX):
            i = NIDX * k + u
            pi = u % NIDX
            pr = u % NBUF

            rows_wait(pr)
            pltpu.sync_copy(rws[pr], acc.at[ibs[pi]], add=True)

            @pl.when(i + NIDX < NCH)
            def _(i=i, pi=pi):
                idx_start(i + NIDX, pi)

            @pl.when(i + NBUF < NCH)
            def _(i=i, u=u, pr=pr):
                pn = (u + NBUF) % NIDX
                idx_wait(pn)
                pltpu.async_copy(x_hbm.at[ias[pn]], rws[pr], srs[pr])

        return 0

    with jax.named_scope("agg_edge_loop"):
        lax.fori_loop(0, NCH // NIDX, step_block, 0)
        plsc.subcore_barrier()

    with jax.named_scope("agg_writeout"):
        obase = cid * N_PAD + row0
        pltpu.sync_copy(acc.at[pl.ds(row0, ROWS_PER_TILE)],
                        out_hbm.at[pl.ds(obase, ROWS_PER_TILE)])


def _sc_aggregate(X, ref_a, ref_b):
    mesh = plsc.VectorSubcoreMesh(core_axis_name="c", subcore_axis_name="s",
                                  num_cores=NC, num_subcores=NS)
    f = pl.kernel(
        _sc_aggregate_body,
        out_type=jax.ShapeDtypeStruct((NC * N_PAD, D_FEAT), jnp.float32),
        mesh=mesh,
        scratch_types=(
            [pltpu.VMEM((CHUNK,), jnp.int32)] * (2 * NIDX)
            + [pltpu.VMEM((CHUNK, D_FEAT), jnp.float32)] * NBUF
            + [pltpu.VMEM((ZROWS, D_FEAT), jnp.float32),
               pltpu.VMEM_SHARED((N_PAD, D_FEAT), jnp.float32)]
            + [pltpu.SemaphoreType.DMA] * (NIDX + NBUF)
        ),
    )
    npad = E_PAD - N_EDGES
    ra = jnp.concatenate([ref_a, (jnp.arange(npad, dtype=jnp.int32) * 64)
                          % N_NODES])
    rb = jnp.concatenate([ref_b, N_NODES
                          + (jnp.arange(npad, dtype=jnp.int32) % (N_PAD
                                                                  - N_NODES))])
    return f(X, ra, rb)


def _mlp_body(x_ref, a0_ref, a1_ref, wh_ref, bh_ref, wo_ref, bo_ref, o_ref):
    xa = x_ref[...] + a0_ref[0] + a1_ref[0]
    h = jnp.dot(xa, wh_ref[...], preferred_element_type=jnp.float32)
    h = jnp.maximum(h + bh_ref[...], 0.0)
    o = jnp.dot(h, wo_ref[...], preferred_element_type=jnp.float32)
    o_ref[...] = jnp.maximum(o + bo_ref[...], 0.0)


def _mlp(X, agg3, W_hidden, b_hidden, W_out, b_out):
    R = 2000
    full = lambda i: (0, 0)
    return pl.pallas_call(
        _mlp_body,
        grid=(N_NODES // R,),
        in_specs=[
            pl.BlockSpec((R, D_FEAT), lambda i: (i, 0)),
            pl.BlockSpec((1, R, D_FEAT), lambda i: (0, i, 0)),
            pl.BlockSpec((1, R, D_FEAT), lambda i: (1, i, 0)),
            pl.BlockSpec((D_FEAT, D_FEAT), full),
            pl.BlockSpec((1, D_FEAT), full),
            pl.BlockSpec((D_FEAT, D_FEAT), full),
            pl.BlockSpec((1, D_FEAT), full),
        ],
        out_specs=pl.BlockSpec((R, D_FEAT), lambda i: (i, 0)),
        out_shape=jax.ShapeDtypeStruct((N_NODES, D_FEAT), jnp.float32),
    )(X, agg3, agg3, W_hidden, b_hidden, W_out, b_out)


@jax.jit
def kernel(X, ref_a, ref_b, W_hidden, b_hidden, W_out, b_out):
    ref_a = ref_a.astype(jnp.int32)
    ref_b = ref_b.astype(jnp.int32)
    agg = _sc_aggregate(X, ref_a, ref_b)
    agg3 = agg.reshape(NC, N_PAD, D_FEAT)
    return _mlp(X, agg3, W_hidden, b_hidden.reshape(1, -1),
                W_out, b_out.reshape(1, -1))


# speedup vs baseline: 1.8590x; 1.8590x over previous
import jax
import jax.numpy as jnp
from jax import lax
from jax.experimental import pallas as pl
from jax.experimental.pallas import tpu as pltpu
from jax.experimental.pallas import tpu_sc as plsc

N_NODES = 10000
N_EDGES = 320000
D_FEAT = 128

NC = 2
NS = 16
NW = NC * NS

N_PAD = 10240
CHUNK = 64
NCH = 160
E_PER_W = NCH * CHUNK
E_PAD = NW * E_PER_W
NBUF = 4
NIDX = 8
ROWS_PER_TILE = N_PAD // NS
ZROWS = 16


def _sc_aggregate_body(x_hbm, ra_hbm, rb_hbm, out_hbm, *refs):
    ias = refs[0:NIDX]
    ibs = refs[NIDX:2 * NIDX]
    rws = refs[2 * NIDX:2 * NIDX + NBUF]
    zbuf = refs[2 * NIDX + NBUF]
    acc = refs[2 * NIDX + NBUF + 1]
    sis = refs[2 * NIDX + NBUF + 2:3 * NIDX + NBUF + 2]
    srs = refs[3 * NIDX + NBUF + 2:3 * NIDX + 2 * NBUF + 2]

    cid = lax.axis_index("c")
    sid = lax.axis_index("s")
    wid = cid * NS + sid
    ebase = wid * E_PER_W

    def idx_start(i, p):
        base = ebase + i * CHUNK
        pltpu.async_copy(ra_hbm.at[pl.ds(base, CHUNK)], ias[p], sis[p])
        pltpu.async_copy(rb_hbm.at[pl.ds(base, CHUNK)], ibs[p], sis[p])

    def idx_wait(p):
        pltpu.make_async_copy(ra_hbm.at[pl.ds(0, CHUNK)], ias[p], sis[p]).wait()
        pltpu.make_async_copy(rb_hbm.at[pl.ds(0, CHUNK)], ibs[p], sis[p]).wait()

    def rows_wait(p):
        pltpu.make_async_copy(x_hbm.at[pl.ds(0, CHUNK)], rws[p], srs[p]).wait()

    for p in range(NIDX):
        idx_start(p, p)

    with jax.named_scope("agg_zero_init"):
        for r in range(ZROWS):
            for j in range(8):
                zbuf[r, pl.ds(j * 16, 16)] = jnp.zeros((16,), jnp.float32)

        row0 = sid * ROWS_PER_TILE

        def zero_acc(k, _):
            pltpu.sync_copy(zbuf, acc.at[pl.ds(row0 + k * ZROWS, ZROWS)])
            return 0

        lax.fori_loop(0, ROWS_PER_TILE // ZROWS, zero_acc, 0)

        plsc.subcore_barrier()

    for j in range(NBUF):
        idx_wait(j)
        pltpu.async_copy(x_hbm.at[ias[j]], rws[j], srs[j])

    def step_block(k, _):
        for u in range(NIDX):
            i = NIDX * k + u
            pi = u % NIDX
            pr = u % NBUF

            rows_wait(pr)
            pltpu.sync_copy(rws[pr], acc.at[ibs[pi]], add=True)

            @pl.when(i + NIDX < NCH)
            def _(i=i, pi=pi):
                idx_start(i + NIDX, pi)

            @pl.when(i + NBUF < NCH)
            def _(i=i, u=u, pr=pr):
                pn = (u + NBUF) % NIDX
                idx_wait(pn)
                pltpu.async_copy(x_hbm.at[ias[pn]], rws[pr], srs[pr])

        return 0

    with jax.named_scope("agg_edge_loop"):
        lax.fori_loop(0, NCH // NIDX, step_block, 0)
        plsc.subcore_barrier()

    with jax.named_scope("agg_writeout"):
        obase = cid * N_PAD + row0
        pltpu.sync_copy(acc.at[pl.ds(row0, ROWS_PER_TILE)],
                        out_hbm.at[pl.ds(obase, ROWS_PER_TILE)])


def _sc_aggregate(X, ref_a, ref_b):
    mesh = plsc.VectorSubcoreMesh(core_axis_name="c", subcore_axis_name="s",
                                  num_cores=NC, num_subcores=NS)
    f = pl.kernel(
        _sc_aggregate_body,
        out_type=jax.ShapeDtypeStruct((NC * N_PAD, D_FEAT), jnp.float32),
        mesh=mesh,
        scratch_types=(
            [pltpu.VMEM((CHUNK,), jnp.int32)] * (2 * NIDX)
            + [pltpu.VMEM((CHUNK, D_FEAT), jnp.float32)] * NBUF
            + [pltpu.VMEM((ZROWS, D_FEAT), jnp.float32),
               pltpu.VMEM_SHARED((N_PAD, D_FEAT), jnp.float32)]
            + [pltpu.SemaphoreType.DMA] * (NIDX + NBUF)
        ),
    )
    npad = E_PAD - N_EDGES
    ra = jnp.concatenate([ref_a, (jnp.arange(npad, dtype=jnp.int32) * 64)
                          % N_NODES])
    rb = jnp.concatenate([ref_b, N_NODES
                          + (jnp.arange(npad, dtype=jnp.int32) % (N_PAD
                                                                  - N_NODES))])
    return f(X, ra, rb)


def _mlp_body(x_ref, a0_ref, a1_ref, wh_ref, bh_ref, wo_ref, bo_ref, o_ref):
    xa = x_ref[...] + a0_ref[0] + a1_ref[0]
    h = jnp.dot(xa, wh_ref[...], preferred_element_type=jnp.float32)
    h = jnp.maximum(h + bh_ref[...], 0.0)
    o = jnp.dot(h, wo_ref[...], preferred_element_type=jnp.float32)
    o_ref[...] = jnp.maximum(o + bo_ref[...], 0.0)


def _mlp(X, agg3, W_hidden, b_hidden, W_out, b_out):
    R = 2000
    full = lambda i: (0, 0)
    return pl.pallas_call(
        _mlp_body,
        grid=(N_NODES // R,),
        in_specs=[
            pl.BlockSpec((R, D_FEAT), lambda i: (i, 0)),
            pl.BlockSpec((1, R, D_FEAT), lambda i: (0, i, 0)),
            pl.BlockSpec((1, R, D_FEAT), lambda i: (1, i, 0)),
            pl.BlockSpec((D_FEAT, D_FEAT), full),
            pl.BlockSpec((1, D_FEAT), full),
            pl.BlockSpec((D_FEAT, D_FEAT), full),
            pl.BlockSpec((1, D_FEAT), full),
        ],
        out_specs=pl.BlockSpec((R, D_FEAT), lambda i: (i, 0)),
        out_shape=jax.ShapeDtypeStruct((N_NODES, D_FEAT), jnp.float32),
    )(X, agg3, agg3, W_hidden, b_hidden, W_out, b_out)


@jax.jit
def kernel(X, ref_a, ref_b, W_hidden, b_hidden, W_out, b_out):
    ref_a = ref_a.astype(jnp.int32)
    ref_b = ref_b.astype(jnp.int32)
    agg = _sc_aggregate(X, ref_a, ref_b)
    agg3 = agg.reshape(NC, N_PAD, D_FEAT)
    return _mlp(X, agg3, W_hidden, b_hidden.reshape(1, -1),
                W_out, b_out.reshape(1, -1))


# speedup vs baseline: 1.8596x; 1.0003x over previous
import jax
import jax.numpy as jnp
from jax import lax
from jax.experimental import pallas as pl
from jax.experimental.pallas import tpu as pltpu
from jax.experimental.pallas import tpu_sc as plsc

N_NODES = 10000
N_EDGES = 320000
D_FEAT = 128

NC = 2
NS = 16
NW = NC * NS

N_PAD = 10240
CHUNK = 80
NCH = 128
E_PER_W = NCH * CHUNK
E_PAD = NW * E_PER_W
NBUF = 4
NIDX = 8
ROWS_PER_TILE = N_PAD // NS
ZROWS = 16


def _sc_aggregate_body(x_hbm, ra_hbm, rb_hbm, out_hbm, *refs):
    ias = refs[0:NIDX]
    ibs = refs[NIDX:2 * NIDX]
    rws = refs[2 * NIDX:2 * NIDX + NBUF]
    zbuf = refs[2 * NIDX + NBUF]
    acc = refs[2 * NIDX + NBUF + 1]
    sis = refs[2 * NIDX + NBUF + 2:3 * NIDX + NBUF + 2]
    srs = refs[3 * NIDX + NBUF + 2:3 * NIDX + 2 * NBUF + 2]

    cid = lax.axis_index("c")
    sid = lax.axis_index("s")
    wid = cid * NS + sid
    ebase = wid * E_PER_W

    def idx_start(i, p):
        base = ebase + i * CHUNK
        pltpu.async_copy(ra_hbm.at[pl.ds(base, CHUNK)], ias[p], sis[p])
        pltpu.async_copy(rb_hbm.at[pl.ds(base, CHUNK)], ibs[p], sis[p])

    def idx_wait(p):
        pltpu.make_async_copy(ra_hbm.at[pl.ds(0, CHUNK)], ias[p], sis[p]).wait()
        pltpu.make_async_copy(rb_hbm.at[pl.ds(0, CHUNK)], ibs[p], sis[p]).wait()

    def rows_wait(p):
        pltpu.make_async_copy(x_hbm.at[pl.ds(0, CHUNK)], rws[p], srs[p]).wait()

    for p in range(NIDX):
        idx_start(p, p)

    with jax.named_scope("agg_zero_init"):
        for r in range(ZROWS):
            for j in range(8):
                zbuf[r, pl.ds(j * 16, 16)] = jnp.zeros((16,), jnp.float32)

        row0 = sid * ROWS_PER_TILE

        def zero_acc(k, _):
            pltpu.sync_copy(zbuf, acc.at[pl.ds(row0 + k * ZROWS, ZROWS)])
            return 0

        lax.fori_loop(0, ROWS_PER_TILE // ZROWS, zero_acc, 0)

        plsc.subcore_barrier()

    for j in range(NBUF):
        idx_wait(j)
        pltpu.async_copy(x_hbm.at[ias[j]], rws[j], srs[j])

    def step_block(k, _):
        for u in range(NIDX):
            i = NIDX * k + u
            pi = u % NIDX
            pr = u % NBUF

            rows_wait(pr)
            pltpu.sync_copy(rws[pr], acc.at[ibs[pi]], add=True)

            @pl.when(i + NIDX < NCH)
            def _(i=i, pi=pi):
                idx_start(i + NIDX, pi)

            @pl.when(i + NBUF < NCH)
            def _(i=i, u=u, pr=pr):
                pn = (u + NBUF) % NIDX
                idx_wait(pn)
                pltpu.async_copy(x_hbm.at[ias[pn]], rws[pr], srs[pr])

        return 0

    with jax.named_scope("agg_edge_loop"):
        lax.fori_loop(0, NCH // NIDX, step_block, 0)
        plsc.subcore_barrier()

    with jax.named_scope("agg_writeout"):
        obase = cid * N_PAD + row0
        pltpu.sync_copy(acc.at[pl.ds(row0, ROWS_PER_TILE)],
                        out_hbm.at[pl.ds(obase, ROWS_PER_TILE)])


def _sc_aggregate(X, ref_a, ref_b):
    mesh = plsc.VectorSubcoreMesh(core_axis_name="c", subcore_axis_name="s",
                                  num_cores=NC, num_subcores=NS)
    f = pl.kernel(
        _sc_aggregate_body,
        out_type=jax.ShapeDtypeStruct((NC * N_PAD, D_FEAT), jnp.float32),
        mesh=mesh,
        scratch_types=(
            [pltpu.VMEM((CHUNK,), jnp.int32)] * (2 * NIDX)
            + [pltpu.VMEM((CHUNK, D_FEAT), jnp.float32)] * NBUF
            + [pltpu.VMEM((ZROWS, D_FEAT), jnp.float32),
               pltpu.VMEM_SHARED((N_PAD, D_FEAT), jnp.float32)]
            + [pltpu.SemaphoreType.DMA] * (NIDX + NBUF)
        ),
    )
    npad = E_PAD - N_EDGES
    ra = jnp.concatenate([ref_a, (jnp.arange(npad, dtype=jnp.int32) * 64)
                          % N_NODES])
    rb = jnp.concatenate([ref_b, N_NODES
                          + (jnp.arange(npad, dtype=jnp.int32) % (N_PAD
                                                                  - N_NODES))])
    return f(X, ra, rb)


def _mlp_body(x_ref, a0_ref, a1_ref, wh_ref, bh_ref, wo_ref, bo_ref, o_ref):
    xa = x_ref[...] + a0_ref[0] + a1_ref[0]
    h = jnp.dot(xa, wh_ref[...], preferred_element_type=jnp.float32)
    h = jnp.maximum(h + bh_ref[...], 0.0)
    o = jnp.dot(h, wo_ref[...], preferred_element_type=jnp.float32)
    o_ref[...] = jnp.maximum(o + bo_ref[...], 0.0)


def _mlp(X, agg3, W_hidden, b_hidden, W_out, b_out):
    R = 2000
    full = lambda i: (0, 0)
    return pl.pallas_call(
        _mlp_body,
        grid=(N_NODES // R,),
        in_specs=[
            pl.BlockSpec((R, D_FEAT), lambda i: (i, 0)),
            pl.BlockSpec((1, R, D_FEAT), lambda i: (0, i, 0)),
            pl.BlockSpec((1, R, D_FEAT), lambda i: (1, i, 0)),
            pl.BlockSpec((D_FEAT, D_FEAT), full),
            pl.BlockSpec((1, D_FEAT), full),
            pl.BlockSpec((D_FEAT, D_FEAT), full),
            pl.BlockSpec((1, D_FEAT), full),
        ],
        out_specs=pl.BlockSpec((R, D_FEAT), lambda i: (i, 0)),
        out_shape=jax.ShapeDtypeStruct((N_NODES, D_FEAT), jnp.float32),
    )(X, agg3, agg3, W_hidden, b_hidden, W_out, b_out)


@jax.jit
def kernel(X, ref_a, ref_b, W_hidden, b_hidden, W_out, b_out):
    ref_a = ref_a.astype(jnp.int32)
    ref_b = ref_b.astype(jnp.int32)
    agg = _sc_aggregate(X, ref_a, ref_b)
    agg3 = agg.reshape(NC, N_PAD, D_FEAT)
    return _mlp(X, agg3, W_hidden, b_hidden.reshape(1, -1),
                W_out, b_out.reshape(1, -1))


# speedup vs baseline: 1.8763x; 1.0090x over previous
import jax
import jax.numpy as jnp
from jax import lax
from jax.experimental import pallas as pl
from jax.experimental.pallas import tpu as pltpu
from jax.experimental.pallas import tpu_sc as plsc

N_NODES = 10000
N_EDGES = 320000
D_FEAT = 128

NC = 2
NS = 16
NW = NC * NS

N_PAD = 10240
CHUNK = 64
NCH = 160
E_PER_W = NCH * CHUNK
E_PAD = NW * E_PER_W
NBUF = 4
NIDX = 8
ROWS_PER_TILE = N_PAD // NS
ZROWS = 32


def _sc_aggregate_body(x_hbm, ra_hbm, rb_hbm, out_hbm, *refs):
    ias = refs[0:NIDX]
    ibs = refs[NIDX:2 * NIDX]
    rws = refs[2 * NIDX:2 * NIDX + NBUF]
    zbuf = refs[2 * NIDX + NBUF]
    acc = refs[2 * NIDX + NBUF + 1]
    sis = refs[2 * NIDX + NBUF + 2:3 * NIDX + NBUF + 2]
    srs = refs[3 * NIDX + NBUF + 2:3 * NIDX + 2 * NBUF + 2]
    szero = refs[3 * NIDX + 2 * NBUF + 2]

    cid = lax.axis_index("c")
    sid = lax.axis_index("s")
    wid = cid * NS + sid
    ebase = wid * E_PER_W

    def idx_start(i, p):
        base = ebase + i * CHUNK
        pltpu.async_copy(ra_hbm.at[pl.ds(base, CHUNK)], ias[p], sis[p])
        pltpu.async_copy(rb_hbm.at[pl.ds(base, CHUNK)], ibs[p], sis[p])

    def idx_wait(p):
        pltpu.make_async_copy(ra_hbm.at[pl.ds(0, CHUNK)], ias[p], sis[p]).wait()
        pltpu.make_async_copy(rb_hbm.at[pl.ds(0, CHUNK)], ibs[p], sis[p]).wait()

    def rows_wait(p):
        pltpu.make_async_copy(x_hbm.at[pl.ds(0, CHUNK)], rws[p], srs[p]).wait()

    for p in range(NIDX):
        idx_start(p, p)

    with jax.named_scope("agg_zero_init"):
        for r in range(ZROWS):
            for j in range(8):
                zbuf[r, pl.ds(j * 16, 16)] = jnp.zeros((16,), jnp.float32)

        row0 = sid * ROWS_PER_TILE
        nzc = ROWS_PER_TILE // ZROWS
        for k in range(nzc):
            pltpu.async_copy(zbuf, acc.at[pl.ds(row0 + k * ZROWS, ZROWS)],
                             szero)
        for k in range(nzc):
            pltpu.make_async_copy(zbuf, acc.at[pl.ds(row0, ZROWS)],
                                  szero).wait()

        plsc.subcore_barrier()

    for j in range(NBUF):
        idx_wait(j)
        pltpu.async_copy(x_hbm.at[ias[j]], rws[j], srs[j])

    def step_block(k, _):
        for u in range(NIDX):
            i = NIDX * k + u
            pi = u % NIDX
            pr = u % NBUF

            rows_wait(pr)
            pltpu.sync_copy(rws[pr], acc.at[ibs[pi]], add=True)

            @pl.when(i + NIDX < NCH)
            def _(i=i, pi=pi):
                idx_start(i + NIDX, pi)

            @pl.when(i + NBUF < NCH)
            def _(i=i, u=u, pr=pr):
                pn = (u + NBUF) % NIDX
                idx_wait(pn)
                pltpu.async_copy(x_hbm.at[ias[pn]], rws[pr], srs[pr])

        return 0

    with jax.named_scope("agg_edge_loop"):
        lax.fori_loop(0, NCH // NIDX, step_block, 0)
        plsc.subcore_barrier()

    with jax.named_scope("agg_writeout"):
        obase = cid * N_PAD + row0
        pltpu.sync_copy(acc.at[pl.ds(row0, ROWS_PER_TILE)],
                        out_hbm.at[pl.ds(obase, ROWS_PER_TILE)])


def _sc_aggregate(X, ref_a, ref_b):
    mesh = plsc.VectorSubcoreMesh(core_axis_name="c", subcore_axis_name="s",
                                  num_cores=NC, num_subcores=NS)
    f = pl.kernel(
        _sc_aggregate_body,
        out_type=jax.ShapeDtypeStruct((NC * N_PAD, D_FEAT), jnp.float32),
        mesh=mesh,
        scratch_types=(
            [pltpu.VMEM((CHUNK,), jnp.int32)] * (2 * NIDX)
            + [pltpu.VMEM((CHUNK, D_FEAT), jnp.float32)] * NBUF
            + [pltpu.VMEM((ZROWS, D_FEAT), jnp.float32),
               pltpu.VMEM_SHARED((N_PAD, D_FEAT), jnp.float32)]
            + [pltpu.SemaphoreType.DMA] * (NIDX + NBUF + 1)
        ),
    )
    npad = E_PAD - N_EDGES
    ra = jnp.concatenate([ref_a, (jnp.arange(npad, dtype=jnp.int32) * 64)
                          % N_NODES])
    rb = jnp.concatenate([ref_b, N_NODES
                          + (jnp.arange(npad, dtype=jnp.int32) % (N_PAD
                                                                  - N_NODES))])
    return f(X, ra, rb)


def _mlp_body(x_ref, a0_ref, a1_ref, wh_ref, bh_ref, wo_ref, bo_ref, o_ref):
    xa = x_ref[...] + a0_ref[0] + a1_ref[0]
    h = jnp.dot(xa, wh_ref[...], preferred_element_type=jnp.float32)
    h = jnp.maximum(h + bh_ref[...], 0.0)
    o = jnp.dot(h, wo_ref[...], preferred_element_type=jnp.float32)
    o_ref[...] = jnp.maximum(o + bo_ref[...], 0.0)


def _mlp(X, agg3, W_hidden, b_hidden, W_out, b_out):
    R = 2000
    full = lambda i: (0, 0)
    return pl.pallas_call(
        _mlp_body,
        grid=(N_NODES // R,),
        in_specs=[
            pl.BlockSpec((R, D_FEAT), lambda i: (i, 0)),
            pl.BlockSpec((1, R, D_FEAT), lambda i: (0, i, 0)),
            pl.BlockSpec((1, R, D_FEAT), lambda i: (1, i, 0)),
            pl.BlockSpec((D_FEAT, D_FEAT), full),
            pl.BlockSpec((1, D_FEAT), full),
            pl.BlockSpec((D_FEAT, D_FEAT), full),
            pl.BlockSpec((1, D_FEAT), full),
        ],
        out_specs=pl.BlockSpec((R, D_FEAT), lambda i: (i, 0)),
        out_shape=jax.ShapeDtypeStruct((N_NODES, D_FEAT), jnp.float32),
    )(X, agg3, agg3, W_hidden, b_hidden, W_out, b_out)


@jax.jit
def kernel(X, ref_a, ref_b, W_hidden, b_hidden, W_out, b_out):
    ref_a = ref_a.astype(jnp.int32)
    ref_b = ref_b.astype(jnp.int32)
    agg = _sc_aggregate(X, ref_a, ref_b)
    agg3 = agg.reshape(NC, N_PAD, D_FEAT)
    return _mlp(X, agg3, W_hidden, b_hidden.reshape(1, -1),
                W_out, b_out.reshape(1, -1))
